# Initial kernel scaffold; baseline (speedup 1.0000x reference)
import jax
import jax.numpy as jnp
from jax.experimental import pallas as pl


def kernel(x, Wg, W1, W2, W3, S1, S2, S3, expert_bias):
    raise NotImplementedError("write your pallas kernel here")


# speedup vs baseline: 4.0061x; 4.0061x over previous
import jax
import jax.numpy as jnp
from jax.experimental import pallas as pl
from jax.experimental.pallas import tpu as pltpu

T = 2048
DIM = 1024
HID = 768
E = 8
TOPK = 2
EPS = 1e-20

BT = 1024
NT = T // BT
NSTEP = E + 1


def _nt_dot(a, b):
    return jax.lax.dot_general(a, b, (((1,), (1,)), ((), ())),
                               preferred_element_type=jnp.float32)


def _moe_body(x_ref, Wg_ref, bias_ref, W1_ref, W2_ref, W3_ref,
              S1_ref, S2_ref, S3_ref, out_ref, gate_ref):
    e = pl.program_id(1)
    x = x_ref[...]
    xb = x.astype(jnp.bfloat16)

    @pl.when(e == 0)
    def _router():
        logits = _nt_dot(xb, Wg_ref[...].astype(jnp.bfloat16))
        scores = jax.nn.sigmoid(logits)
        biased = scores + bias_ref[...]
        iota = jax.lax.broadcasted_iota(jnp.int32, (BT, E), 1)
        m1 = jnp.argmax(biased, axis=1)[:, None]
        sel1 = iota == m1
        m2 = jnp.argmax(jnp.where(sel1, -jnp.inf, biased), axis=1)[:, None]
        sel2 = iota == m2
        w1 = jnp.sum(jnp.where(sel1, scores, 0.0), axis=1, keepdims=True)
        w2 = jnp.sum(jnp.where(sel2, scores, 0.0), axis=1, keepdims=True)
        denom = w1 + w2 + EPS
        gate_ref[...] = (jnp.where(sel1, w1, 0.0)
                         + jnp.where(sel2, w2, 0.0)) / denom

    def swiglu(w1w, w3w, w2w):
        a = _nt_dot(xb, w1w.astype(jnp.bfloat16))
        b = _nt_dot(xb, w3w.astype(jnp.bfloat16))
        h = (a * jax.nn.sigmoid(a) * b).astype(jnp.bfloat16)
        return _nt_dot(h, w2w.astype(jnp.bfloat16))

    @pl.when(e < E)
    def _routed():
        oe = swiglu(W1_ref[0], W3_ref[0], W2_ref[0])
        iota = jax.lax.broadcasted_iota(jnp.int32, (BT, E), 1)
        g = jnp.sum(jnp.where(iota == e, gate_ref[...], 0.0),
                    axis=1, keepdims=True)
        contrib = oe * g

        @pl.when(e == 0)
        def _():
            out_ref[...] = contrib

        @pl.when(e > 0)
        def _():
            out_ref[...] += contrib

    @pl.when(e == E)
    def _shared():
        out_ref[...] += swiglu(S1_ref[...], S3_ref[...], S2_ref[...])


def kernel(x, Wg, W1, W2, W3, S1, S2, S3, expert_bias):
    bias2 = expert_bias.reshape(1, E)
    we_idx = lambda t, e: (jnp.minimum(e, E - 1), 0, 0)
    out = pl.pallas_call(
        _moe_body,
        grid=(NT, NSTEP),
        in_specs=[
            pl.BlockSpec((BT, DIM), lambda t, e: (t, 0)),
            pl.BlockSpec((E, DIM), lambda t, e: (0, 0)),
            pl.BlockSpec((1, E), lambda t, e: (0, 0)),
            pl.BlockSpec((1, HID, DIM), we_idx),
            pl.BlockSpec((1, DIM, HID), we_idx),
            pl.BlockSpec((1, HID, DIM), we_idx),
            pl.BlockSpec((HID, DIM), lambda t, e: (0, 0)),
            pl.BlockSpec((DIM, HID), lambda t, e: (0, 0)),
            pl.BlockSpec((HID, DIM), lambda t, e: (0, 0)),
        ],
        out_specs=pl.BlockSpec((BT, DIM), lambda t, e: (t, 0)),
        out_shape=jax.ShapeDtypeStruct((T, DIM), jnp.float32),
        scratch_shapes=[pltpu.VMEM((BT, E), jnp.float32)],
    )(x, Wg, bias2, W1, W2, W3, S1, S2, S3)
    return out
